# baseline (speedup 1.0000x reference)
import functools

import numpy as np
import jax
import jax.numpy as jnp
from jax import lax
from jax.experimental import pallas as pl
from jax.experimental.pallas import tpu as pltpu
from jax.experimental.pallas import tpu_sc as plsc

SEQ_LEN = 2048
VOCAB = 100000
D_MODEL = 1024
BATCH = 4

NW = 32
S_PER_W = SEQ_LEN // NW
CHUNK = 16
NCH = S_PER_W // CHUNK
NUNITS = NCH * BATCH
LANES = 16
VPR = D_MODEL // LANES


def _pos_encoding() -> np.ndarray:
    pos = np.arange(SEQ_LEN)[:, None].astype(np.float32)
    i = np.arange(D_MODEL)[None, :]
    angle_rates = 1.0 / np.power(10000.0, (2.0 * (i // 2)) / np.float32(D_MODEL))
    angles = pos * angle_rates
    return np.where(i % 2 == 0, np.sin(angles), np.cos(angles)).astype(np.float32)


_PE = _pos_encoding()


_MESH = plsc.VectorSubcoreMesh(core_axis_name="c", subcore_axis_name="s")


@functools.partial(
    pl.kernel,
    mesh=_MESH,
    out_type=[
        jax.ShapeDtypeStruct((BATCH, SEQ_LEN, D_MODEL), jnp.float32),
        jax.ShapeDtypeStruct((BATCH, SEQ_LEN, D_MODEL), jnp.float32),
    ],
    scratch_types=[
        pltpu.VMEM((BATCH * S_PER_W,), jnp.int32),
        pltpu.VMEM((BATCH * S_PER_W,), jnp.int32),
        pltpu.VMEM((2, CHUNK, D_MODEL), jnp.float32),
        pltpu.VMEM((2, CHUNK, D_MODEL), jnp.float32),
        pltpu.VMEM((2, CHUNK, D_MODEL), jnp.float32),
        pltpu.SemaphoreType.DMA((2,)),
        pltpu.SemaphoreType.DMA((2,)),
        pltpu.SemaphoreType.DMA((2,)),
        pltpu.SemaphoreType.DMA((2,)),
        pltpu.SemaphoreType.DMA,
    ],
)
def _emb_kernel(x_hbm, xo_hbm, pe_hbm, tab_hbm, enc_hbm, dec_hbm,
                idx_e, idx_d, emb_e, emb_d, pe_v,
                sem_ge, sem_gd, sem_we, sem_wd, sem_pe):
    wid = lax.axis_index("s") * 2 + lax.axis_index("c")
    s0 = wid * S_PER_W

    for b in range(BATCH):
        pltpu.sync_copy(x_hbm.at[pl.ds(b * SEQ_LEN + s0, S_PER_W)],
                        idx_e.at[pl.ds(b * S_PER_W, S_PER_W)])
        pltpu.sync_copy(xo_hbm.at[pl.ds(b * SEQ_LEN + s0, S_PER_W)],
                        idx_d.at[pl.ds(b * S_PER_W, S_PER_W)])

    pltpu.async_copy(pe_hbm.at[pl.ds(s0, CHUNK)], pe_v.at[0], sem_pe)
    pltpu.async_copy(tab_hbm.at[idx_e.at[pl.ds(0, CHUNK)]],
                     emb_e.at[0], sem_ge.at[0])
    pltpu.async_copy(tab_hbm.at[idx_d.at[pl.ds(0, CHUNK)]],
                     emb_d.at[0], sem_gd.at[0])

    def unit_body(u, carry):
        b = lax.rem(u, BATCH)
        c = lax.div(u, BATCH)
        cur = lax.rem(u, 2)
        nxt = 1 - cur
        pc = lax.rem(c, 2)
        sb = s0 + c * CHUNK

        @pl.when(b == 0)
        def _():
            pltpu.make_async_copy(pe_hbm.at[pl.ds(s0, CHUNK)],
                                  pe_v.at[pc], sem_pe).wait()

            @pl.when(c + 1 < NCH)
            def _():
                pltpu.async_copy(pe_hbm.at[pl.ds(sb + CHUNK, CHUNK)],
                                 pe_v.at[1 - pc], sem_pe)

        @pl.when(u >= 1)
        def _():
            pltpu.make_async_copy(emb_e.at[nxt],
                                  enc_hbm.at[0, pl.ds(0, CHUNK)],
                                  sem_we.at[nxt]).wait()
            pltpu.make_async_copy(emb_d.at[nxt],
                                  dec_hbm.at[0, pl.ds(0, CHUNK)],
                                  sem_wd.at[nxt]).wait()

        @pl.when(u + 1 < NUNITS)
        def _():
            u1 = u + 1
            b1 = lax.rem(u1, BATCH)
            c1 = lax.div(u1, BATCH)
            off1 = b1 * S_PER_W + c1 * CHUNK
            pltpu.async_copy(tab_hbm.at[idx_e.at[pl.ds(off1, CHUNK)]],
                             emb_e.at[nxt], sem_ge.at[nxt])
            pltpu.async_copy(tab_hbm.at[idx_d.at[pl.ds(off1, CHUNK)]],
                             emb_d.at[nxt], sem_gd.at[nxt])

        off = b * S_PER_W + c * CHUNK
        pltpu.make_async_copy(tab_hbm.at[idx_e.at[pl.ds(off, CHUNK)]],
                              emb_e.at[cur], sem_ge.at[cur]).wait()
        pltpu.make_async_copy(tab_hbm.at[idx_d.at[pl.ds(off, CHUNK)]],
                              emb_d.at[cur], sem_gd.at[cur]).wait()

        def row_body(r, rcarry):
            for j in range(VPR):
                sl = pl.ds(j * LANES, LANES)
                pv = pe_v[pc, r, sl]
                emb_e[cur, r, sl] += pv
                emb_d[cur, r, sl] += pv
            return rcarry

        lax.fori_loop(0, CHUNK, row_body, 0)

        pltpu.async_copy(emb_e.at[cur], enc_hbm.at[b, pl.ds(sb, CHUNK)],
                         sem_we.at[cur])
        pltpu.async_copy(emb_d.at[cur], dec_hbm.at[b, pl.ds(sb, CHUNK)],
                         sem_wd.at[cur])
        return carry

    lax.fori_loop(0, NUNITS, unit_body, 0)

    last = (NUNITS - 1) % 2
    pltpu.make_async_copy(emb_e.at[last], enc_hbm.at[0, pl.ds(0, CHUNK)],
                          sem_we.at[last]).wait()
    pltpu.make_async_copy(emb_d.at[last], dec_hbm.at[0, pl.ds(0, CHUNK)],
                          sem_wd.at[last]).wait()


def kernel(x, x_output, emb_table):
    enc, dec = _emb_kernel(x.reshape(-1), x_output.reshape(-1),
                           jnp.asarray(_PE), emb_table)
    return (enc, dec)


# speedup vs baseline: 1.2814x; 1.2814x over previous
import functools

import numpy as np
import jax
import jax.numpy as jnp
from jax import lax
from jax.experimental import pallas as pl
from jax.experimental.pallas import tpu as pltpu
from jax.experimental.pallas import tpu_sc as plsc

SEQ_LEN = 2048
VOCAB = 100000
D_MODEL = 1024
BATCH = 4

NW = 32
S_PER_W = SEQ_LEN // NW
CHUNK = 16
NCH = S_PER_W // CHUNK
NUNITS = NCH * BATCH
NBUF = 3
LANES = 16
VPR = D_MODEL // LANES


def _pos_encoding() -> np.ndarray:
    pos = np.arange(SEQ_LEN)[:, None].astype(np.float32)
    i = np.arange(D_MODEL)[None, :]
    angle_rates = 1.0 / np.power(10000.0, (2.0 * (i // 2)) / np.float32(D_MODEL))
    angles = pos * angle_rates
    return np.where(i % 2 == 0, np.sin(angles), np.cos(angles)).astype(np.float32)


_PE = _pos_encoding()


_MESH = plsc.VectorSubcoreMesh(core_axis_name="c", subcore_axis_name="s")


@functools.partial(
    pl.kernel,
    mesh=_MESH,
    out_type=[
        jax.ShapeDtypeStruct((BATCH, SEQ_LEN, D_MODEL), jnp.float32),
        jax.ShapeDtypeStruct((BATCH, SEQ_LEN, D_MODEL), jnp.float32),
    ],
    scratch_types=[
        pltpu.VMEM((BATCH * S_PER_W,), jnp.int32),
        pltpu.VMEM((BATCH * S_PER_W,), jnp.int32),
        pltpu.VMEM((NBUF, CHUNK, D_MODEL), jnp.float32),
        pltpu.VMEM((NBUF, CHUNK, D_MODEL), jnp.float32),
        pltpu.VMEM((CHUNK, D_MODEL), jnp.float32),
        pltpu.SemaphoreType.DMA((NBUF,)),
        pltpu.SemaphoreType.DMA((NBUF,)),
        pltpu.SemaphoreType.DMA((NBUF,)),
        pltpu.SemaphoreType.DMA((NBUF,)),
    ],
)
def _emb_kernel(x_hbm, xo_hbm, pe_hbm, tab_hbm, enc_hbm, dec_hbm,
                idx_e, idx_d, emb_e, emb_d, pe_v,
                sem_ge, sem_gd, sem_we, sem_wd):
    wid = lax.axis_index("s") * 2 + lax.axis_index("c")
    s0 = wid * S_PER_W

    for b in range(BATCH):
        pltpu.sync_copy(x_hbm.at[pl.ds(b * SEQ_LEN + s0, S_PER_W)],
                        idx_e.at[pl.ds(b * S_PER_W, S_PER_W)])
        pltpu.sync_copy(xo_hbm.at[pl.ds(b * SEQ_LEN + s0, S_PER_W)],
                        idx_d.at[pl.ds(b * S_PER_W, S_PER_W)])

    pltpu.async_copy(tab_hbm.at[idx_e.at[pl.ds(0, CHUNK)]],
                     emb_e.at[0], sem_ge.at[0])
    pltpu.async_copy(tab_hbm.at[idx_d.at[pl.ds(0, CHUNK)]],
                     emb_d.at[0], sem_gd.at[0])

    def unit_body(u, carry):
        b = lax.rem(u, BATCH)
        c = lax.div(u, BATCH)
        cur = lax.rem(u, NBUF)
        nx = lax.rem(u + 1, NBUF)
        sb = s0 + c * CHUNK

        @pl.when(b == 0)
        def _():
            pltpu.sync_copy(pe_hbm.at[pl.ds(sb, CHUNK)], pe_v)

        @pl.when(u >= NBUF - 1)
        def _():
            pltpu.make_async_copy(emb_e.at[nx],
                                  enc_hbm.at[0, pl.ds(0, CHUNK)],
                                  sem_we.at[nx]).wait()
            pltpu.make_async_copy(emb_d.at[nx],
                                  dec_hbm.at[0, pl.ds(0, CHUNK)],
                                  sem_wd.at[nx]).wait()

        @pl.when(u + 1 < NUNITS)
        def _():
            u1 = u + 1
            off1 = lax.rem(u1, BATCH) * S_PER_W + lax.div(u1, BATCH) * CHUNK
            pltpu.async_copy(tab_hbm.at[idx_e.at[pl.ds(off1, CHUNK)]],
                             emb_e.at[nx], sem_ge.at[nx])
            pltpu.async_copy(tab_hbm.at[idx_d.at[pl.ds(off1, CHUNK)]],
                             emb_d.at[nx], sem_gd.at[nx])

        off = b * S_PER_W + c * CHUNK
        pltpu.make_async_copy(tab_hbm.at[idx_e.at[pl.ds(off, CHUNK)]],
                              emb_e.at[cur], sem_ge.at[cur]).wait()
        pltpu.make_async_copy(tab_hbm.at[idx_d.at[pl.ds(off, CHUNK)]],
                              emb_d.at[cur], sem_gd.at[cur]).wait()

        def row_body(r, rcarry):
            for j in range(VPR):
                sl = pl.ds(j * LANES, LANES)
                pv = pe_v[r, sl]
                plsc.addupdate(emb_e.at[cur, r, sl], pv)
                plsc.addupdate(emb_d.at[cur, r, sl], pv)
            return rcarry

        lax.fori_loop(0, CHUNK, row_body, 0)

        pltpu.async_copy(emb_e.at[cur], enc_hbm.at[b, pl.ds(sb, CHUNK)],
                         sem_we.at[cur])
        pltpu.async_copy(emb_d.at[cur], dec_hbm.at[b, pl.ds(sb, CHUNK)],
                         sem_wd.at[cur])
        return carry

    lax.fori_loop(0, NUNITS, unit_body, 0)

    for u in (NUNITS - 2, NUNITS - 1):
        s = u % NBUF
        pltpu.make_async_copy(emb_e.at[s], enc_hbm.at[0, pl.ds(0, CHUNK)],
                              sem_we.at[s]).wait()
        pltpu.make_async_copy(emb_d.at[s], dec_hbm.at[0, pl.ds(0, CHUNK)],
                              sem_wd.at[s]).wait()


def kernel(x, x_output, emb_table):
    enc, dec = _emb_kernel(x.reshape(-1), x_output.reshape(-1),
                           jnp.asarray(_PE), emb_table)
    return (enc, dec)


# speedup vs baseline: 1.6708x; 1.3039x over previous
import functools

import numpy as np
import jax
import jax.numpy as jnp
from jax import lax
from jax.experimental import pallas as pl
from jax.experimental.pallas import tpu as pltpu
from jax.experimental.pallas import tpu_sc as plsc

SEQ_LEN = 2048
VOCAB = 100000
D_MODEL = 1024
BATCH = 4

NW = 32
S_PER_W = SEQ_LEN // NW
CHUNK = 16
NCH = S_PER_W // CHUNK
NUNITS = NCH * BATCH
NBUF = 3
LANES = 16
VPR = D_MODEL // LANES


def _pos_encoding() -> np.ndarray:
    pos = np.arange(SEQ_LEN)[:, None].astype(np.float32)
    i = np.arange(D_MODEL)[None, :]
    angle_rates = 1.0 / np.power(10000.0, (2.0 * (i // 2)) / np.float32(D_MODEL))
    angles = pos * angle_rates
    return np.where(i % 2 == 0, np.sin(angles), np.cos(angles)).astype(np.float32)


_PE = _pos_encoding()


_MESH = plsc.VectorSubcoreMesh(core_axis_name="c", subcore_axis_name="s")


@functools.partial(
    pl.kernel,
    mesh=_MESH,
    out_type=[
        jax.ShapeDtypeStruct((BATCH, SEQ_LEN, D_MODEL), jnp.float32),
        jax.ShapeDtypeStruct((BATCH, SEQ_LEN, D_MODEL), jnp.float32),
    ],
    scratch_types=[
        pltpu.VMEM((BATCH * S_PER_W,), jnp.int32),
        pltpu.VMEM((BATCH * S_PER_W,), jnp.int32),
        pltpu.VMEM((NBUF, CHUNK, D_MODEL), jnp.float32),
        pltpu.VMEM((NBUF, CHUNK, D_MODEL), jnp.float32),
        pltpu.VMEM((CHUNK, D_MODEL), jnp.float32),
        pltpu.SemaphoreType.DMA((NBUF,)),
        pltpu.SemaphoreType.DMA((NBUF,)),
        pltpu.SemaphoreType.DMA((NBUF,)),
        pltpu.SemaphoreType.DMA((NBUF,)),
    ],
)
def _emb_kernel(x_hbm, xo_hbm, pe_hbm, tab_hbm, enc_hbm, dec_hbm,
                idx_e, idx_d, emb_e, emb_d, pe_v,
                sem_ge, sem_gd, sem_we, sem_wd):
    wid = lax.axis_index("s") * 2 + lax.axis_index("c")
    s0 = wid * S_PER_W

    for b in range(BATCH):
        pltpu.sync_copy(x_hbm.at[pl.ds(b * SEQ_LEN + s0, S_PER_W)],
                        idx_e.at[pl.ds(b * S_PER_W, S_PER_W)])
        pltpu.sync_copy(xo_hbm.at[pl.ds(b * SEQ_LEN + s0, S_PER_W)],
                        idx_d.at[pl.ds(b * S_PER_W, S_PER_W)])

    pltpu.async_copy(tab_hbm.at[idx_e.at[pl.ds(0, CHUNK)]],
                     emb_e.at[0], sem_ge.at[0])
    pltpu.async_copy(tab_hbm.at[idx_d.at[pl.ds(0, CHUNK)]],
                     emb_d.at[0], sem_gd.at[0])

    def unit_body(u, carry):
        b = lax.rem(u, BATCH)
        c = lax.div(u, BATCH)
        cur = lax.rem(u, NBUF)
        nx = lax.rem(u + 1, NBUF)
        sb = s0 + c * CHUNK

        @pl.when(b == 0)
        def _():
            pltpu.sync_copy(pe_hbm.at[pl.ds(sb, CHUNK)], pe_v)

        @pl.when(u >= NBUF - 1)
        def _():
            pltpu.make_async_copy(emb_e.at[nx],
                                  enc_hbm.at[0, pl.ds(0, CHUNK)],
                                  sem_we.at[nx]).wait()
            pltpu.make_async_copy(emb_d.at[nx],
                                  dec_hbm.at[0, pl.ds(0, CHUNK)],
                                  sem_wd.at[nx]).wait()

        @pl.when(u + 1 < NUNITS)
        def _():
            u1 = u + 1
            off1 = lax.rem(u1, BATCH) * S_PER_W + lax.div(u1, BATCH) * CHUNK
            pltpu.async_copy(tab_hbm.at[idx_e.at[pl.ds(off1, CHUNK)]],
                             emb_e.at[nx], sem_ge.at[nx])
            pltpu.async_copy(tab_hbm.at[idx_d.at[pl.ds(off1, CHUNK)]],
                             emb_d.at[nx], sem_gd.at[nx])

        off = b * S_PER_W + c * CHUNK
        pltpu.make_async_copy(tab_hbm.at[idx_e.at[pl.ds(off, CHUNK)]],
                              emb_e.at[cur], sem_ge.at[cur]).wait()
        pltpu.make_async_copy(tab_hbm.at[idx_d.at[pl.ds(off, CHUNK)]],
                              emb_d.at[cur], sem_gd.at[cur]).wait()

        @plsc.parallel_loop(0, CHUNK, unroll=2)
        def row_body(r):
            for j in range(VPR):
                sl = pl.ds(j * LANES, LANES)
                pv = pe_v[r, sl]
                plsc.addupdate(emb_e.at[cur, r, sl], pv)
                plsc.addupdate(emb_d.at[cur, r, sl], pv)

        pltpu.async_copy(emb_e.at[cur], enc_hbm.at[b, pl.ds(sb, CHUNK)],
                         sem_we.at[cur])
        pltpu.async_copy(emb_d.at[cur], dec_hbm.at[b, pl.ds(sb, CHUNK)],
                         sem_wd.at[cur])
        return carry

    lax.fori_loop(0, NUNITS, unit_body, 0)

    for u in (NUNITS - 2, NUNITS - 1):
        s = u % NBUF
        pltpu.make_async_copy(emb_e.at[s], enc_hbm.at[0, pl.ds(0, CHUNK)],
                              sem_we.at[s]).wait()
        pltpu.make_async_copy(emb_d.at[s], dec_hbm.at[0, pl.ds(0, CHUNK)],
                              sem_wd.at[s]).wait()


def kernel(x, x_output, emb_table):
    enc, dec = _emb_kernel(x.reshape(-1), x_output.reshape(-1),
                           jnp.asarray(_PE), emb_table)
    return (enc, dec)
